# baseline (speedup 1.0000x reference)
import functools

import jax
import jax.numpy as jnp
from jax import lax
from jax.experimental import pallas as pl
from jax.experimental.pallas import tpu as pltpu
from jax.experimental.pallas import tpu_sc as plsc

VOCAB = 1000000
EMBED = 32
PROMPT = 128
BATCH = 16384
SEQ = 200

NC, NS = 2, 16
NW = NC * NS
B_PER_W = BATCH // NW
CB = 8
IDX_SLICE = 40
G_PER_CHUNK = CB * SEQ // IDX_SLICE
N_CHUNKS = B_PER_W // CB
IDX_ROWS_PER_W = B_PER_W * SEQ // IDX_SLICE
UNROLL = 8
INV_SEQ = 1.0 / SEQ


def _sc_gather_pool(ids2d, table):
    mesh = plsc.VectorSubcoreMesh(core_axis_name="c", subcore_axis_name="s")

    @functools.partial(
        pl.kernel,
        out_type=jax.ShapeDtypeStruct((BATCH, EMBED), jnp.float32),
        mesh=mesh,
        scratch_types=[
            pltpu.VMEM((3, G_PER_CHUNK, IDX_SLICE), jnp.int32),
            pltpu.VMEM((2, CB * SEQ, EMBED), jnp.float32),
            pltpu.VMEM((B_PER_W, EMBED), jnp.float32),
            pltpu.SemaphoreType.DMA,
            pltpu.SemaphoreType.DMA,
            pltpu.SemaphoreType.DMA,
            pltpu.SemaphoreType.DMA,
            pltpu.SemaphoreType.DMA,
        ],
        compiler_params=pltpu.CompilerParams(use_tc_tiling_on_sc=False),
    )
    def k(ids_hbm, table_hbm, out_hbm, idx_v, rows_v, pooled_v,
          sem0, sem1, isem0, isem1, isem2):
        wid = lax.axis_index("s") * NC + lax.axis_index("c")
        row0 = wid * IDX_ROWS_PER_W
        bbase = wid * B_PER_W
        sems = (sem0, sem1)
        isems = (isem0, isem1, isem2)
        last_row = (NW * IDX_ROWS_PER_W) - G_PER_CHUNK

        def idx_src(g):
            return ids_hbm.at[pl.ds(
                jnp.minimum(row0 + g * G_PER_CHUNK, last_row), G_PER_CHUNK)]

        def idx_fire(g, ibuf):
            pltpu.async_copy(idx_src(g), idx_v.at[ibuf], isems[ibuf])

        def idx_wait(g, ibuf):
            pltpu.make_async_copy(idx_src(g), idx_v.at[ibuf],
                                  isems[ibuf]).wait()

        def fire(ibuf, buf):
            for j in range(G_PER_CHUNK):
                pltpu.async_copy(
                    table_hbm.at[idx_v.at[ibuf, j]],
                    rows_v.at[buf, pl.ds(j * IDX_SLICE, IDX_SLICE)],
                    sems[buf])

        def drain(ibuf, buf):
            for j in range(G_PER_CHUNK):
                pltpu.make_async_copy(
                    table_hbm.at[idx_v.at[ibuf, j]],
                    rows_v.at[buf, pl.ds(j * IDX_SLICE, IDX_SLICE)],
                    sems[buf]).wait()

        def reduce_chunk(g, buf):
            for b in range(CB):
                zero = jnp.zeros((16,), jnp.float32)
                npairs = 4

                def body(i, accs, _b=b, _buf=buf):
                    accs = list(accs)
                    for u in range(UNROLL):
                        r = _b * SEQ + i * UNROLL + u
                        a = 2 * (u % npairs)
                        accs[a] = accs[a] + rows_v[_buf, r, pl.ds(0, 16)]
                        accs[a + 1] = (
                            accs[a + 1] + rows_v[_buf, r, pl.ds(16, 16)])
                    return tuple(accs)

                accs = lax.fori_loop(0, SEQ // UNROLL, body,
                                     (zero,) * (2 * npairs))
                a0 = (accs[0] + accs[2]) + (accs[4] + accs[6])
                a1 = (accs[1] + accs[3]) + (accs[5] + accs[7])
                pooled_v[g * CB + b, pl.ds(0, 16)] = a0 * INV_SEQ
                pooled_v[g * CB + b, pl.ds(16, 16)] = a1 * INV_SEQ

        assert (N_CHUNKS - 4) % 6 == 0
        idx_fire(0, 0)
        idx_fire(1, 1)
        idx_wait(0, 0)
        fire(0, 0)
        idx_fire(2, 2)

        @pl.loop(1, N_CHUNKS - 3, step=6)
        def _(go):
            for p in range(6):
                g = go + p
                rb = (1 + p) % 2
                ib = (1 + p) % 3
                ibm = (p) % 3
                idx_wait(g, ib)
                fire(ib, rb)
                drain(ibm, 1 - rb)
                idx_fire(g + 2, ibm)
                reduce_chunk(g - 1, 1 - rb)

        for g in range(N_CHUNKS - 3, N_CHUNKS):
            rb, ib, ibm = g % 2, g % 3, (g - 1) % 3
            idx_wait(g, ib)
            fire(ib, rb)
            drain(ibm, 1 - rb)
            if g + 2 < N_CHUNKS:
                idx_fire(g + 2, ibm)
            reduce_chunk(g - 1, 1 - rb)
        drain((N_CHUNKS - 1) % 3, (N_CHUNKS - 1) % 2)
        reduce_chunk(N_CHUNKS - 1, (N_CHUNKS - 1) % 2)

        pltpu.sync_copy(pooled_v, out_hbm.at[pl.ds(bbase, B_PER_W)])

    return k(ids2d, table)


N_CBLK = VOCAB // 128
CBLK_PER_W = N_CBLK // NW
TAIL_V = N_CBLK * 128


def _sc_format(tableT, tail_lin):
    mesh = plsc.VectorSubcoreMesh(core_axis_name="c", subcore_axis_name="s")

    @functools.partial(
        pl.kernel,
        out_type=jax.ShapeDtypeStruct((VOCAB // 4, 128), jnp.float32),
        mesh=mesh,
        scratch_types=[
            pltpu.VMEM((2, 4, 8, 128), jnp.float32),
            pltpu.VMEM((2, 32, 128), jnp.float32),
            pltpu.VMEM((8, 8, EMBED), jnp.float32),
            pltpu.SemaphoreType.DMA,
            pltpu.SemaphoreType.DMA,
            pltpu.SemaphoreType.DMA,
            pltpu.SemaphoreType.DMA,
        ],
        compiler_params=pltpu.CompilerParams(use_tc_tiling_on_sc=True,
                                             needs_layout_passes=False),
    )
    def k(tab_hbm, tail_hbm, fmt_hbm, in_v, out_v, tail_v,
          is0, is1, os0, os1):
        wid = lax.axis_index("s") * NC + lax.axis_index("c")
        cb0 = wid * CBLK_PER_W
        isems = (is0, is1)
        osems = (os0, os1)
        iota = lax.iota(jnp.int32, 16)

        def in_pairs(cb, buf):
            return [(tab_hbm.at[pl.ds(8 * a, 8), pl.ds(cb * 128, 128)],
                     in_v.at[buf, a]) for a in range(4)]

        def in_fire(cb, buf):
            for s, d in in_pairs(cb, buf):
                pltpu.async_copy(s, d, isems[buf])

        def in_wait(cb, buf):
            for s, d in in_pairs(cb, buf):
                pltpu.make_async_copy(s, d, isems[buf]).wait()

        def out_fire(cb, buf):
            pltpu.async_copy(out_v.at[buf],
                             fmt_hbm.at[pl.ds(cb * 32, 32)], osems[buf])

        def out_wait(cb, buf):
            pltpu.make_async_copy(out_v.at[buf],
                                  fmt_hbm.at[pl.ds(cb * 32, 32)],
                                  osems[buf]).wait()

        def transpose(buf):
            src = in_v.at[buf]
            for r in range(128):
                for h in range(2):
                    vals = plsc.load_gather(
                        src,
                        [2 * h + iota // 8, iota % 8,
                         jnp.full((16,), r, jnp.int32)])
                    out_v[buf, r // 4,
                          pl.ds(32 * (r % 4) + 16 * h, 16)] = vals

        in_fire(cb0, 0)
        in_fire(cb0 + 1, 1)
        for p in range(2):
            in_wait(cb0 + p, p)
            transpose(p)
            out_fire(cb0 + p, p)
            in_fire(cb0 + p + 2, p)

        @pl.loop(2, CBLK_PER_W - 2, step=2)
        def _(go):
            for p in range(2):
                cb = cb0 + go + p
                in_wait(cb, p)
                out_wait(cb - 2, p)
                transpose(p)
                out_fire(cb, p)
                in_fire(cb + 2, p)

        for p in range(2):
            cb = cb0 + CBLK_PER_W - 2 + p
            in_wait(cb, p)
            out_wait(cb - 2, p)
            transpose(p)
            out_fire(cb, p)
        for p in range(2):
            out_wait(cb0 + CBLK_PER_W - 2 + p, p)

        @pl.when(wid < 4)
        def _():
            cb = NW * CBLK_PER_W + wid
            for s, d in in_pairs(cb, 0):
                pltpu.sync_copy(s, d)
            transpose(0)
            pltpu.sync_copy(out_v.at[0], fmt_hbm.at[pl.ds(cb * 32, 32)])

        @pl.when(wid == 4)
        def _():
            for a in range(8):
                pltpu.sync_copy(tail_hbm.at[pl.ds(8 * a, 8)], tail_v.at[a])
            for r in range(64):
                for h in range(2):
                    out_v[0, r // 4, pl.ds(32 * (r % 4) + 16 * h, 16)] = (
                        tail_v[r // 8, r % 8, pl.ds(16 * h, 16)])
            pltpu.sync_copy(out_v.at[0, pl.ds(0, 16)],
                            fmt_hbm.at[pl.ds(TAIL_V // 4, 16)])

    return k(tableT, tail_lin)


def _mlp(pooled, W1, b1, W2, b2):
    BLK = 2048

    def body(p_ref, w1_ref, b1_ref, w2_ref, b2_ref, o_ref):
        h = jnp.dot(p_ref[...], w1_ref[...],
                    preferred_element_type=jnp.float32) + b1_ref[...]
        h = jnp.maximum(h, 0.0)
        o_ref[...] = jnp.dot(h, w2_ref[...],
                             preferred_element_type=jnp.float32) + b2_ref[...]

    return pl.pallas_call(
        body,
        out_shape=jax.ShapeDtypeStruct((BATCH, PROMPT), jnp.float32),
        grid=(BATCH // BLK,),
        in_specs=[
            pl.BlockSpec((BLK, EMBED), lambda i: (i, 0)),
            pl.BlockSpec((EMBED, PROMPT), lambda i: (0, 0)),
            pl.BlockSpec((1, PROMPT), lambda i: (0, 0)),
            pl.BlockSpec((PROMPT, PROMPT), lambda i: (0, 0)),
            pl.BlockSpec((1, PROMPT), lambda i: (0, 0)),
        ],
        out_specs=pl.BlockSpec((BLK, PROMPT), lambda i: (i, 0)),
    )(pooled, W1, b1.reshape(1, PROMPT), W2, b2.reshape(1, PROMPT))


def kernel(prompt_ids, table, W1, b1, W2, b2):
    ids2d = prompt_ids.reshape(BATCH * SEQ // IDX_SLICE, IDX_SLICE)
    tail_lin = lax.slice(table, (TAIL_V, 0), (VOCAB, EMBED))
    table_lin = _sc_format(table.T, tail_lin).reshape(VOCAB, EMBED)
    pooled = _sc_gather_pool(ids2d, table_lin)
    return _mlp(pooled, W1, b1, W2, b2)


# speedup vs baseline: 1.0494x; 1.0494x over previous
import functools

import jax
import jax.numpy as jnp
from jax import lax
from jax.experimental import pallas as pl
from jax.experimental.pallas import tpu as pltpu
from jax.experimental.pallas import tpu_sc as plsc

VOCAB = 1000000
EMBED = 32
PROMPT = 128
BATCH = 16384
SEQ = 200

NC, NS = 2, 16
NW = NC * NS
B_PER_W = BATCH // NW
CB = 8
IDX_SLICE = 40
G_PER_CHUNK = CB * SEQ // IDX_SLICE
N_CHUNKS = B_PER_W // CB
IDX_ROWS_PER_W = B_PER_W * SEQ // IDX_SLICE
UNROLL = 8
INV_SEQ = 1.0 / SEQ


def _sc_gather_pool(ids2d, table):
    mesh = plsc.VectorSubcoreMesh(core_axis_name="c", subcore_axis_name="s")

    @functools.partial(
        pl.kernel,
        out_type=jax.ShapeDtypeStruct((BATCH, EMBED), jnp.float32),
        mesh=mesh,
        scratch_types=[
            pltpu.VMEM((3, G_PER_CHUNK, IDX_SLICE), jnp.int32),
            pltpu.VMEM((2, CB * SEQ, EMBED), jnp.float32),
            pltpu.VMEM((B_PER_W, EMBED), jnp.float32),
            pltpu.SemaphoreType.DMA,
            pltpu.SemaphoreType.DMA,
            pltpu.SemaphoreType.DMA,
            pltpu.SemaphoreType.DMA,
            pltpu.SemaphoreType.DMA,
        ],
        compiler_params=pltpu.CompilerParams(use_tc_tiling_on_sc=False),
    )
    def k(ids_hbm, table_hbm, out_hbm, idx_v, rows_v, pooled_v,
          sem0, sem1, isem0, isem1, isem2):
        wid = lax.axis_index("s") * NC + lax.axis_index("c")
        row0 = wid * IDX_ROWS_PER_W
        bbase = wid * B_PER_W
        sems = (sem0, sem1)
        isems = (isem0, isem1, isem2)
        last_row = (NW * IDX_ROWS_PER_W) - G_PER_CHUNK

        def idx_src(g):
            return ids_hbm.at[pl.ds(
                jnp.minimum(row0 + g * G_PER_CHUNK, last_row), G_PER_CHUNK)]

        def idx_fire(g, ibuf):
            pltpu.async_copy(idx_src(g), idx_v.at[ibuf], isems[ibuf])

        def idx_wait(g, ibuf):
            pltpu.make_async_copy(idx_src(g), idx_v.at[ibuf],
                                  isems[ibuf]).wait()

        def fire(ibuf, buf):
            for j in range(G_PER_CHUNK):
                pltpu.async_copy(
                    table_hbm.at[idx_v.at[ibuf, j]],
                    rows_v.at[buf, pl.ds(j * IDX_SLICE, IDX_SLICE)],
                    sems[buf])

        def drain(ibuf, buf):
            for j in range(G_PER_CHUNK):
                pltpu.make_async_copy(
                    table_hbm.at[idx_v.at[ibuf, j]],
                    rows_v.at[buf, pl.ds(j * IDX_SLICE, IDX_SLICE)],
                    sems[buf]).wait()

        def reduce_chunk(g, buf):
            for b in range(CB):
                zero = jnp.zeros((16,), jnp.float32)
                npairs = 4

                def body(i, accs, _b=b, _buf=buf):
                    accs = list(accs)
                    for u in range(UNROLL):
                        r = _b * SEQ + i * UNROLL + u
                        a = 2 * (u % npairs)
                        accs[a] = accs[a] + rows_v[_buf, r, pl.ds(0, 16)]
                        accs[a + 1] = (
                            accs[a + 1] + rows_v[_buf, r, pl.ds(16, 16)])
                    return tuple(accs)

                accs = lax.fori_loop(0, SEQ // UNROLL, body,
                                     (zero,) * (2 * npairs))
                a0 = (accs[0] + accs[2]) + (accs[4] + accs[6])
                a1 = (accs[1] + accs[3]) + (accs[5] + accs[7])
                pooled_v[g * CB + b, pl.ds(0, 16)] = a0 * INV_SEQ
                pooled_v[g * CB + b, pl.ds(16, 16)] = a1 * INV_SEQ

        assert (N_CHUNKS - 4) % 6 == 0
        idx_fire(0, 0)
        idx_fire(1, 1)
        idx_wait(0, 0)
        fire(0, 0)
        idx_fire(2, 2)

        @pl.loop(1, N_CHUNKS - 3, step=6)
        def _(go):
            for p in range(6):
                g = go + p
                rb = (1 + p) % 2
                ib = (1 + p) % 3
                ibm = (p) % 3
                idx_wait(g, ib)
                fire(ib, rb)
                drain(ibm, 1 - rb)
                idx_fire(g + 2, ibm)
                reduce_chunk(g - 1, 1 - rb)

        for g in range(N_CHUNKS - 3, N_CHUNKS):
            rb, ib, ibm = g % 2, g % 3, (g - 1) % 3
            idx_wait(g, ib)
            fire(ib, rb)
            drain(ibm, 1 - rb)
            if g + 2 < N_CHUNKS:
                idx_fire(g + 2, ibm)
            reduce_chunk(g - 1, 1 - rb)
        drain((N_CHUNKS - 1) % 3, (N_CHUNKS - 1) % 2)
        reduce_chunk(N_CHUNKS - 1, (N_CHUNKS - 1) % 2)

        pltpu.sync_copy(pooled_v, out_hbm.at[pl.ds(bbase, B_PER_W)])

    return k(ids2d, table)


N_CBLK = VOCAB // 128
CBLK_PER_W = N_CBLK // NW
TAIL_V = N_CBLK * 128


def _sc_format(tableT, tail_lin):
    mesh = plsc.VectorSubcoreMesh(core_axis_name="c", subcore_axis_name="s")

    @functools.partial(
        pl.kernel,
        out_type=jax.ShapeDtypeStruct((VOCAB // 4, 128), jnp.float32),
        mesh=mesh,
        scratch_types=[
            pltpu.VMEM((2, 4, 8, 129), jnp.float32),
            pltpu.VMEM((2, 32, 128), jnp.float32),
            pltpu.VMEM((8, 8, EMBED), jnp.float32),
            pltpu.SemaphoreType.DMA,
            pltpu.SemaphoreType.DMA,
            pltpu.SemaphoreType.DMA,
            pltpu.SemaphoreType.DMA,
        ],
        compiler_params=pltpu.CompilerParams(use_tc_tiling_on_sc=True,
                                             needs_layout_passes=False),
    )
    def k(tab_hbm, tail_hbm, fmt_hbm, in_v, out_v, tail_v,
          is0, is1, os0, os1):
        wid = lax.axis_index("s") * NC + lax.axis_index("c")
        cb0 = wid * CBLK_PER_W
        isems = (is0, is1)
        osems = (os0, os1)
        iota = lax.iota(jnp.int32, 16)

        def in_pairs(cb, buf):
            return [(tab_hbm.at[pl.ds(8 * a, 8), pl.ds(cb * 128, 128)],
                     in_v.at[buf, a, :, pl.ds(0, 128)]) for a in range(4)]

        def in_fire(cb, buf):
            for s, d in in_pairs(cb, buf):
                pltpu.async_copy(s, d, isems[buf])

        def in_wait(cb, buf):
            for s, d in in_pairs(cb, buf):
                pltpu.make_async_copy(s, d, isems[buf]).wait()

        def out_fire(cb, buf):
            pltpu.async_copy(out_v.at[buf],
                             fmt_hbm.at[pl.ds(cb * 32, 32)], osems[buf])

        def out_wait(cb, buf):
            pltpu.make_async_copy(out_v.at[buf],
                                  fmt_hbm.at[pl.ds(cb * 32, 32)],
                                  osems[buf]).wait()

        def transpose(buf):
            src = in_v.at[buf]

            def body(rb, carry, _buf=buf):
                for u in range(8):
                    r = rb * 8 + u
                    for h in range(2):
                        vals = plsc.load_gather(
                            src,
                            [2 * h + iota // 8, iota % 8,
                             jnp.full((16,), r, jnp.int32)])
                        out_v[_buf, rb * 2 + u // 4,
                              pl.ds(32 * (u % 4) + 16 * h, 16)] = vals
                return carry

            lax.fori_loop(0, 16, body, 0)

        in_fire(cb0, 0)
        in_fire(cb0 + 1, 1)
        for p in range(2):
            in_wait(cb0 + p, p)
            transpose(p)
            out_fire(cb0 + p, p)
            in_fire(cb0 + p + 2, p)

        @pl.loop(2, CBLK_PER_W - 2, step=2)
        def _(go):
            for p in range(2):
                cb = cb0 + go + p
                in_wait(cb, p)
                out_wait(cb - 2, p)
                transpose(p)
                out_fire(cb, p)
                in_fire(cb + 2, p)

        for p in range(2):
            cb = cb0 + CBLK_PER_W - 2 + p
            in_wait(cb, p)
            out_wait(cb - 2, p)
            transpose(p)
            out_fire(cb, p)
        for p in range(2):
            out_wait(cb0 + CBLK_PER_W - 2 + p, p)

        @pl.when(wid < 4)
        def _():
            cb = NW * CBLK_PER_W + wid
            for s, d in in_pairs(cb, 0):
                pltpu.sync_copy(s, d)
            transpose(0)
            pltpu.sync_copy(out_v.at[0], fmt_hbm.at[pl.ds(cb * 32, 32)])

        @pl.when(wid == 4)
        def _():
            for a in range(8):
                pltpu.sync_copy(tail_hbm.at[pl.ds(8 * a, 8)], tail_v.at[a])
            for r in range(64):
                for h in range(2):
                    out_v[0, r // 4, pl.ds(32 * (r % 4) + 16 * h, 16)] = (
                        tail_v[r // 8, r % 8, pl.ds(16 * h, 16)])
            pltpu.sync_copy(out_v.at[0, pl.ds(0, 16)],
                            fmt_hbm.at[pl.ds(TAIL_V // 4, 16)])

    return k(tableT, tail_lin)


def _mlp(pooled, W1, b1, W2, b2):
    BLK = 2048

    def body(p_ref, w1_ref, b1_ref, w2_ref, b2_ref, o_ref):
        h = jnp.dot(p_ref[...], w1_ref[...],
                    preferred_element_type=jnp.float32) + b1_ref[...]
        h = jnp.maximum(h, 0.0)
        o_ref[...] = jnp.dot(h, w2_ref[...],
                             preferred_element_type=jnp.float32) + b2_ref[...]

    return pl.pallas_call(
        body,
        out_shape=jax.ShapeDtypeStruct((BATCH, PROMPT), jnp.float32),
        grid=(BATCH // BLK,),
        in_specs=[
            pl.BlockSpec((BLK, EMBED), lambda i: (i, 0)),
            pl.BlockSpec((EMBED, PROMPT), lambda i: (0, 0)),
            pl.BlockSpec((1, PROMPT), lambda i: (0, 0)),
            pl.BlockSpec((PROMPT, PROMPT), lambda i: (0, 0)),
            pl.BlockSpec((1, PROMPT), lambda i: (0, 0)),
        ],
        out_specs=pl.BlockSpec((BLK, PROMPT), lambda i: (i, 0)),
    )(pooled, W1, b1.reshape(1, PROMPT), W2, b2.reshape(1, PROMPT))


def kernel(prompt_ids, table, W1, b1, W2, b2):
    ids2d = prompt_ids.reshape(BATCH * SEQ // IDX_SLICE, IDX_SLICE)
    tail_lin = lax.slice(table, (TAIL_V, 0), (VOCAB, EMBED))
    table_lin = _sc_format(table.T, tail_lin).reshape(VOCAB, EMBED)
    pooled = _sc_gather_pool(ids2d, table_lin)
    return _mlp(pooled, W1, b1, W2, b2)


# speedup vs baseline: 2.0022x; 1.9079x over previous
import functools

import jax
import jax.numpy as jnp
from jax import lax
from jax.experimental import pallas as pl
from jax.experimental.pallas import tpu as pltpu
from jax.experimental.pallas import tpu_sc as plsc

VOCAB = 1000000
EMBED = 32
PROMPT = 128
BATCH = 16384
SEQ = 200

NC, NS = 2, 16
NW = NC * NS
B_PER_W = BATCH // NW
CB = 8
IDX_SLICE = 40
G_PER_CHUNK = CB * SEQ // IDX_SLICE
N_CHUNKS = B_PER_W // CB
IDX_ROWS_PER_W = B_PER_W * SEQ // IDX_SLICE
UNROLL = 8
INV_SEQ = 1.0 / SEQ


def _sc_gather_pool(ids2d, table):
    mesh = plsc.VectorSubcoreMesh(core_axis_name="c", subcore_axis_name="s")

    @functools.partial(
        pl.kernel,
        out_type=jax.ShapeDtypeStruct((BATCH, EMBED), jnp.float32),
        mesh=mesh,
        scratch_types=[
            pltpu.VMEM((3, G_PER_CHUNK, IDX_SLICE), jnp.int32),
            pltpu.VMEM((2, CB * SEQ, EMBED), jnp.float32),
            pltpu.VMEM((B_PER_W, EMBED), jnp.float32),
            pltpu.SemaphoreType.DMA,
            pltpu.SemaphoreType.DMA,
            pltpu.SemaphoreType.DMA,
            pltpu.SemaphoreType.DMA,
            pltpu.SemaphoreType.DMA,
        ],
        compiler_params=pltpu.CompilerParams(use_tc_tiling_on_sc=False),
    )
    def k(ids_hbm, table_hbm, out_hbm, idx_v, rows_v, pooled_v,
          sem0, sem1, isem0, isem1, isem2):
        wid = lax.axis_index("s") * NC + lax.axis_index("c")
        row0 = wid * IDX_ROWS_PER_W
        bbase = wid * B_PER_W
        sems = (sem0, sem1)
        isems = (isem0, isem1, isem2)
        last_row = (NW * IDX_ROWS_PER_W) - G_PER_CHUNK

        def idx_src(g):
            return ids_hbm.at[pl.ds(
                jnp.minimum(row0 + g * G_PER_CHUNK, last_row), G_PER_CHUNK)]

        def idx_fire(g, ibuf):
            pltpu.async_copy(idx_src(g), idx_v.at[ibuf], isems[ibuf])

        def idx_wait(g, ibuf):
            pltpu.make_async_copy(idx_src(g), idx_v.at[ibuf],
                                  isems[ibuf]).wait()

        def fire(ibuf, buf):
            for j in range(G_PER_CHUNK):
                pltpu.async_copy(
                    table_hbm.at[idx_v.at[ibuf, j]],
                    rows_v.at[buf, pl.ds(j * IDX_SLICE, IDX_SLICE)],
                    sems[buf])

        def drain(ibuf, buf):
            for j in range(G_PER_CHUNK):
                pltpu.make_async_copy(
                    table_hbm.at[idx_v.at[ibuf, j]],
                    rows_v.at[buf, pl.ds(j * IDX_SLICE, IDX_SLICE)],
                    sems[buf]).wait()

        def reduce_chunk(g, buf):
            for b in range(CB):
                zero = jnp.zeros((16,), jnp.float32)
                npairs = 4

                def body(i, accs, _b=b, _buf=buf):
                    accs = list(accs)
                    for u in range(UNROLL):
                        r = _b * SEQ + i * UNROLL + u
                        a = 2 * (u % npairs)
                        accs[a] = accs[a] + rows_v[_buf, r, pl.ds(0, 16)]
                        accs[a + 1] = (
                            accs[a + 1] + rows_v[_buf, r, pl.ds(16, 16)])
                    return tuple(accs)

                accs = lax.fori_loop(0, SEQ // UNROLL, body,
                                     (zero,) * (2 * npairs))
                a0 = (accs[0] + accs[2]) + (accs[4] + accs[6])
                a1 = (accs[1] + accs[3]) + (accs[5] + accs[7])
                pooled_v[g * CB + b, pl.ds(0, 16)] = a0 * INV_SEQ
                pooled_v[g * CB + b, pl.ds(16, 16)] = a1 * INV_SEQ

        assert (N_CHUNKS - 4) % 6 == 0
        idx_fire(0, 0)
        idx_fire(1, 1)
        idx_wait(0, 0)
        fire(0, 0)
        idx_fire(2, 2)

        @pl.loop(1, N_CHUNKS - 3, step=6)
        def _(go):
            for p in range(6):
                g = go + p
                rb = (1 + p) % 2
                ib = (1 + p) % 3
                ibm = (p) % 3
                idx_wait(g, ib)
                fire(ib, rb)
                drain(ibm, 1 - rb)
                idx_fire(g + 2, ibm)
                reduce_chunk(g - 1, 1 - rb)

        for g in range(N_CHUNKS - 3, N_CHUNKS):
            rb, ib, ibm = g % 2, g % 3, (g - 1) % 3
            idx_wait(g, ib)
            fire(ib, rb)
            drain(ibm, 1 - rb)
            if g + 2 < N_CHUNKS:
                idx_fire(g + 2, ibm)
            reduce_chunk(g - 1, 1 - rb)
        drain((N_CHUNKS - 1) % 3, (N_CHUNKS - 1) % 2)
        reduce_chunk(N_CHUNKS - 1, (N_CHUNKS - 1) % 2)

        pltpu.sync_copy(pooled_v, out_hbm.at[pl.ds(bbase, B_PER_W)])

    return k(ids2d, table)


def _format_table(tableT):
    VB = 4096
    SUB = 4
    VS = VB // SUB

    def body(x_ref, o_ref, t_scr):
        for s in range(SUB):
            t_scr[pl.ds(s * VS, VS), :] = jnp.transpose(
                x_ref[:, pl.ds(s * VS, VS)], (1, 0))
        for s in range(SUB):
            o_ref[pl.ds(s * VS // 4, VS // 4), :] = jnp.concatenate(
                [t_scr[pl.Slice(s * VS + q, VS // 4, 4), :]
                 for q in range(4)], axis=1)

    return pl.pallas_call(
        body,
        out_shape=jax.ShapeDtypeStruct((VOCAB // 4, 128), jnp.float32),
        grid=(pl.cdiv(VOCAB, VB),),
        in_specs=[pl.BlockSpec((EMBED, VB), lambda i: (0, i))],
        out_specs=pl.BlockSpec((VB // 4, 128), lambda i: (i, 0)),
        scratch_shapes=[pltpu.VMEM((VB, EMBED), jnp.float32)],
    )(tableT)


N_CBLK = VOCAB // 128
CBLK_PER_W = N_CBLK // NW
TAIL_V = N_CBLK * 128


def _sc_format(tableT, tail_lin):
    mesh = plsc.VectorSubcoreMesh(core_axis_name="c", subcore_axis_name="s")

    @functools.partial(
        pl.kernel,
        out_type=jax.ShapeDtypeStruct((VOCAB // 4, 128), jnp.float32),
        mesh=mesh,
        scratch_types=[
            pltpu.VMEM((2, 4, 8, 129), jnp.float32),
            pltpu.VMEM((2, 32, 128), jnp.float32),
            pltpu.VMEM((8, 8, EMBED), jnp.float32),
            pltpu.SemaphoreType.DMA,
            pltpu.SemaphoreType.DMA,
            pltpu.SemaphoreType.DMA,
            pltpu.SemaphoreType.DMA,
        ],
        compiler_params=pltpu.CompilerParams(use_tc_tiling_on_sc=True,
                                             needs_layout_passes=False),
    )
    def k(tab_hbm, tail_hbm, fmt_hbm, in_v, out_v, tail_v,
          is0, is1, os0, os1):
        wid = lax.axis_index("s") * NC + lax.axis_index("c")
        cb0 = wid * CBLK_PER_W
        isems = (is0, is1)
        osems = (os0, os1)
        iota = lax.iota(jnp.int32, 16)

        def in_pairs(cb, buf):
            return [(tab_hbm.at[pl.ds(8 * a, 8), pl.ds(cb * 128, 128)],
                     in_v.at[buf, a, :, pl.ds(0, 128)]) for a in range(4)]

        def in_fire(cb, buf):
            for s, d in in_pairs(cb, buf):
                pltpu.async_copy(s, d, isems[buf])

        def in_wait(cb, buf):
            for s, d in in_pairs(cb, buf):
                pltpu.make_async_copy(s, d, isems[buf]).wait()

        def out_fire(cb, buf):
            pltpu.async_copy(out_v.at[buf],
                             fmt_hbm.at[pl.ds(cb * 32, 32)], osems[buf])

        def out_wait(cb, buf):
            pltpu.make_async_copy(out_v.at[buf],
                                  fmt_hbm.at[pl.ds(cb * 32, 32)],
                                  osems[buf]).wait()

        def transpose(buf):
            src = in_v.at[buf]

            def body(rb, carry, _buf=buf):
                for u in range(8):
                    r = rb * 8 + u
                    for h in range(2):
                        vals = plsc.load_gather(
                            src,
                            [2 * h + iota // 8, iota % 8,
                             jnp.full((16,), r, jnp.int32)])
                        out_v[_buf, rb * 2 + u // 4,
                              pl.ds(32 * (u % 4) + 16 * h, 16)] = vals
                return carry

            lax.fori_loop(0, 16, body, 0)

        in_fire(cb0, 0)
        in_fire(cb0 + 1, 1)
        for p in range(2):
            in_wait(cb0 + p, p)
            transpose(p)
            out_fire(cb0 + p, p)
            in_fire(cb0 + p + 2, p)

        @pl.loop(2, CBLK_PER_W - 2, step=2)
        def _(go):
            for p in range(2):
                cb = cb0 + go + p
                in_wait(cb, p)
                out_wait(cb - 2, p)
                transpose(p)
                out_fire(cb, p)
                in_fire(cb + 2, p)

        for p in range(2):
            cb = cb0 + CBLK_PER_W - 2 + p
            in_wait(cb, p)
            out_wait(cb - 2, p)
            transpose(p)
            out_fire(cb, p)
        for p in range(2):
            out_wait(cb0 + CBLK_PER_W - 2 + p, p)

        @pl.when(wid < 4)
        def _():
            cb = NW * CBLK_PER_W + wid
            for s, d in in_pairs(cb, 0):
                pltpu.sync_copy(s, d)
            transpose(0)
            pltpu.sync_copy(out_v.at[0], fmt_hbm.at[pl.ds(cb * 32, 32)])

        @pl.when(wid == 4)
        def _():
            for a in range(8):
                pltpu.sync_copy(tail_hbm.at[pl.ds(8 * a, 8)], tail_v.at[a])
            for r in range(64):
                for h in range(2):
                    out_v[0, r // 4, pl.ds(32 * (r % 4) + 16 * h, 16)] = (
                        tail_v[r // 8, r % 8, pl.ds(16 * h, 16)])
            pltpu.sync_copy(out_v.at[0, pl.ds(0, 16)],
                            fmt_hbm.at[pl.ds(TAIL_V // 4, 16)])

    return k(tableT, tail_lin)


def _mlp(pooled, W1, b1, W2, b2):
    BLK = 2048

    def body(p_ref, w1_ref, b1_ref, w2_ref, b2_ref, o_ref):
        h = jnp.dot(p_ref[...], w1_ref[...],
                    preferred_element_type=jnp.float32) + b1_ref[...]
        h = jnp.maximum(h, 0.0)
        o_ref[...] = jnp.dot(h, w2_ref[...],
                             preferred_element_type=jnp.float32) + b2_ref[...]

    return pl.pallas_call(
        body,
        out_shape=jax.ShapeDtypeStruct((BATCH, PROMPT), jnp.float32),
        grid=(BATCH // BLK,),
        in_specs=[
            pl.BlockSpec((BLK, EMBED), lambda i: (i, 0)),
            pl.BlockSpec((EMBED, PROMPT), lambda i: (0, 0)),
            pl.BlockSpec((1, PROMPT), lambda i: (0, 0)),
            pl.BlockSpec((PROMPT, PROMPT), lambda i: (0, 0)),
            pl.BlockSpec((1, PROMPT), lambda i: (0, 0)),
        ],
        out_specs=pl.BlockSpec((BLK, PROMPT), lambda i: (i, 0)),
    )(pooled, W1, b1.reshape(1, PROMPT), W2, b2.reshape(1, PROMPT))


def kernel(prompt_ids, table, W1, b1, W2, b2):
    ids2d = prompt_ids.reshape(BATCH * SEQ // IDX_SLICE, IDX_SLICE)
    table_lin = _format_table(table.T).reshape(VOCAB, EMBED)
    pooled = _sc_gather_pool(ids2d, table_lin)
    return _mlp(pooled, W1, b1, W2, b2)


# speedup vs baseline: 2.1972x; 1.0974x over previous
import functools

import jax
import jax.numpy as jnp
from jax import lax
from jax.experimental import pallas as pl
from jax.experimental.pallas import tpu as pltpu
from jax.experimental.pallas import tpu_sc as plsc

VOCAB = 1000000
EMBED = 32
PROMPT = 128
BATCH = 16384
SEQ = 200

NC, NS = 2, 16
NW = NC * NS
B_PER_W = BATCH // NW
CB = 8
IDX_SLICE = 40
G_PER_CHUNK = CB * SEQ // IDX_SLICE
N_CHUNKS = B_PER_W // CB
IDX_ROWS_PER_W = B_PER_W * SEQ // IDX_SLICE
UNROLL = 8
INV_SEQ = 1.0 / SEQ


def _sc_gather_pool(ids2d, table):
    mesh = plsc.VectorSubcoreMesh(core_axis_name="c", subcore_axis_name="s")

    @functools.partial(
        pl.kernel,
        out_type=jax.ShapeDtypeStruct((BATCH, EMBED), jnp.float32),
        mesh=mesh,
        scratch_types=[
            pltpu.VMEM((3, G_PER_CHUNK, IDX_SLICE), jnp.int32),
            pltpu.VMEM((2, CB * SEQ, EMBED), jnp.float32),
            pltpu.VMEM((B_PER_W, EMBED), jnp.float32),
            pltpu.SemaphoreType.DMA,
            pltpu.SemaphoreType.DMA,
            pltpu.SemaphoreType.DMA,
            pltpu.SemaphoreType.DMA,
            pltpu.SemaphoreType.DMA,
        ],
        compiler_params=pltpu.CompilerParams(use_tc_tiling_on_sc=False),
    )
    def k(ids_hbm, table_hbm, out_hbm, idx_v, rows_v, pooled_v,
          sem0, sem1, isem0, isem1, isem2):
        wid = lax.axis_index("s") * NC + lax.axis_index("c")
        row0 = wid * IDX_ROWS_PER_W
        bbase = wid * B_PER_W
        sems = (sem0, sem1)
        isems = (isem0, isem1, isem2)
        last_row = (NW * IDX_ROWS_PER_W) - G_PER_CHUNK

        def idx_src(g):
            return ids_hbm.at[pl.ds(
                jnp.minimum(row0 + g * G_PER_CHUNK, last_row), G_PER_CHUNK)]

        def idx_fire(g, ibuf):
            pltpu.async_copy(idx_src(g), idx_v.at[ibuf], isems[ibuf])

        def idx_wait(g, ibuf):
            pltpu.make_async_copy(idx_src(g), idx_v.at[ibuf],
                                  isems[ibuf]).wait()

        def fire(ibuf, buf):
            for j in range(G_PER_CHUNK):
                pltpu.async_copy(
                    table_hbm.at[idx_v.at[ibuf, j]],
                    rows_v.at[buf, pl.ds(j * IDX_SLICE, IDX_SLICE)],
                    sems[buf])

        def drain(ibuf, buf):
            for j in range(G_PER_CHUNK):
                pltpu.make_async_copy(
                    table_hbm.at[idx_v.at[ibuf, j]],
                    rows_v.at[buf, pl.ds(j * IDX_SLICE, IDX_SLICE)],
                    sems[buf]).wait()

        def reduce_chunk(g, buf):
            for b in range(CB):
                zero = jnp.zeros((16,), jnp.float32)
                npairs = 4

                def body(i, accs, _b=b, _buf=buf):
                    accs = list(accs)
                    for u in range(UNROLL):
                        r = _b * SEQ + i * UNROLL + u
                        a = 2 * (u % npairs)
                        accs[a] = accs[a] + rows_v[_buf, r, pl.ds(0, 16)]
                        accs[a + 1] = (
                            accs[a + 1] + rows_v[_buf, r, pl.ds(16, 16)])
                    return tuple(accs)

                accs = lax.fori_loop(0, SEQ // UNROLL, body,
                                     (zero,) * (2 * npairs))
                a0 = (accs[0] + accs[2]) + (accs[4] + accs[6])
                a1 = (accs[1] + accs[3]) + (accs[5] + accs[7])
                pooled_v[g * CB + b, pl.ds(0, 16)] = a0 * INV_SEQ
                pooled_v[g * CB + b, pl.ds(16, 16)] = a1 * INV_SEQ

        assert (N_CHUNKS - 4) % 6 == 0
        idx_fire(0, 0)
        idx_fire(1, 1)
        idx_wait(0, 0)
        fire(0, 0)
        idx_fire(2, 2)

        @pl.loop(1, N_CHUNKS - 3, step=6)
        def _(go):
            for p in range(6):
                g = go + p
                rb = (1 + p) % 2
                ib = (1 + p) % 3
                ibm = (p) % 3
                idx_wait(g, ib)
                fire(ib, rb)
                drain(ibm, 1 - rb)
                idx_fire(g + 2, ibm)
                reduce_chunk(g - 1, 1 - rb)

        for g in range(N_CHUNKS - 3, N_CHUNKS):
            rb, ib, ibm = g % 2, g % 3, (g - 1) % 3
            idx_wait(g, ib)
            fire(ib, rb)
            drain(ibm, 1 - rb)
            if g + 2 < N_CHUNKS:
                idx_fire(g + 2, ibm)
            reduce_chunk(g - 1, 1 - rb)
        drain((N_CHUNKS - 1) % 3, (N_CHUNKS - 1) % 2)
        reduce_chunk(N_CHUNKS - 1, (N_CHUNKS - 1) % 2)

        pltpu.sync_copy(pooled_v, out_hbm.at[pl.ds(bbase, B_PER_W)])

    return k(ids2d, table)


def _format_table(tableT):
    VB = 8192
    SUB = 8
    VS = VB // SUB

    def body(x_ref, o_ref, t_scr):
        for s in range(SUB):
            t_scr[pl.ds(s * VS, VS), :] = jnp.transpose(
                x_ref[:, pl.ds(s * VS, VS)], (1, 0))
        for s in range(SUB):
            o_ref[pl.ds(s * VS // 4, VS // 4), :] = jnp.concatenate(
                [t_scr[pl.Slice(s * VS + q, VS // 4, 4), :]
                 for q in range(4)], axis=1)

    return pl.pallas_call(
        body,
        out_shape=jax.ShapeDtypeStruct((VOCAB // 4, 128), jnp.float32),
        grid=(pl.cdiv(VOCAB, VB),),
        in_specs=[pl.BlockSpec((EMBED, VB), lambda i: (0, i))],
        out_specs=pl.BlockSpec((VB // 4, 128), lambda i: (i, 0)),
        scratch_shapes=[pltpu.VMEM((VB, EMBED), jnp.float32)],
    )(tableT)


N_CBLK = VOCAB // 128
CBLK_PER_W = N_CBLK // NW
TAIL_V = N_CBLK * 128


def _sc_format(tableT, tail_lin):
    mesh = plsc.VectorSubcoreMesh(core_axis_name="c", subcore_axis_name="s")

    @functools.partial(
        pl.kernel,
        out_type=jax.ShapeDtypeStruct((VOCAB // 4, 128), jnp.float32),
        mesh=mesh,
        scratch_types=[
            pltpu.VMEM((2, 4, 8, 129), jnp.float32),
            pltpu.VMEM((2, 32, 128), jnp.float32),
            pltpu.VMEM((8, 8, EMBED), jnp.float32),
            pltpu.SemaphoreType.DMA,
            pltpu.SemaphoreType.DMA,
            pltpu.SemaphoreType.DMA,
            pltpu.SemaphoreType.DMA,
        ],
        compiler_params=pltpu.CompilerParams(use_tc_tiling_on_sc=True,
                                             needs_layout_passes=False),
    )
    def k(tab_hbm, tail_hbm, fmt_hbm, in_v, out_v, tail_v,
          is0, is1, os0, os1):
        wid = lax.axis_index("s") * NC + lax.axis_index("c")
        cb0 = wid * CBLK_PER_W
        isems = (is0, is1)
        osems = (os0, os1)
        iota = lax.iota(jnp.int32, 16)

        def in_pairs(cb, buf):
            return [(tab_hbm.at[pl.ds(8 * a, 8), pl.ds(cb * 128, 128)],
                     in_v.at[buf, a, :, pl.ds(0, 128)]) for a in range(4)]

        def in_fire(cb, buf):
            for s, d in in_pairs(cb, buf):
                pltpu.async_copy(s, d, isems[buf])

        def in_wait(cb, buf):
            for s, d in in_pairs(cb, buf):
                pltpu.make_async_copy(s, d, isems[buf]).wait()

        def out_fire(cb, buf):
            pltpu.async_copy(out_v.at[buf],
                             fmt_hbm.at[pl.ds(cb * 32, 32)], osems[buf])

        def out_wait(cb, buf):
            pltpu.make_async_copy(out_v.at[buf],
                                  fmt_hbm.at[pl.ds(cb * 32, 32)],
                                  osems[buf]).wait()

        def transpose(buf):
            src = in_v.at[buf]

            def body(rb, carry, _buf=buf):
                for u in range(8):
                    r = rb * 8 + u
                    for h in range(2):
                        vals = plsc.load_gather(
                            src,
                            [2 * h + iota // 8, iota % 8,
                             jnp.full((16,), r, jnp.int32)])
                        out_v[_buf, rb * 2 + u // 4,
                              pl.ds(32 * (u % 4) + 16 * h, 16)] = vals
                return carry

            lax.fori_loop(0, 16, body, 0)

        in_fire(cb0, 0)
        in_fire(cb0 + 1, 1)
        for p in range(2):
            in_wait(cb0 + p, p)
            transpose(p)
            out_fire(cb0 + p, p)
            in_fire(cb0 + p + 2, p)

        @pl.loop(2, CBLK_PER_W - 2, step=2)
        def _(go):
            for p in range(2):
                cb = cb0 + go + p
                in_wait(cb, p)
                out_wait(cb - 2, p)
                transpose(p)
                out_fire(cb, p)
                in_fire(cb + 2, p)

        for p in range(2):
            cb = cb0 + CBLK_PER_W - 2 + p
            in_wait(cb, p)
            out_wait(cb - 2, p)
            transpose(p)
            out_fire(cb, p)
        for p in range(2):
            out_wait(cb0 + CBLK_PER_W - 2 + p, p)

        @pl.when(wid < 4)
        def _():
            cb = NW * CBLK_PER_W + wid
            for s, d in in_pairs(cb, 0):
                pltpu.sync_copy(s, d)
            transpose(0)
            pltpu.sync_copy(out_v.at[0], fmt_hbm.at[pl.ds(cb * 32, 32)])

        @pl.when(wid == 4)
        def _():
            for a in range(8):
                pltpu.sync_copy(tail_hbm.at[pl.ds(8 * a, 8)], tail_v.at[a])
            for r in range(64):
                for h in range(2):
                    out_v[0, r // 4, pl.ds(32 * (r % 4) + 16 * h, 16)] = (
                        tail_v[r // 8, r % 8, pl.ds(16 * h, 16)])
            pltpu.sync_copy(out_v.at[0, pl.ds(0, 16)],
                            fmt_hbm.at[pl.ds(TAIL_V // 4, 16)])

    return k(tableT, tail_lin)


def _mlp(pooled, W1, b1, W2, b2):
    BLK = 2048

    def body(p_ref, w1_ref, b1_ref, w2_ref, b2_ref, o_ref):
        h = jnp.dot(p_ref[...], w1_ref[...],
                    preferred_element_type=jnp.float32) + b1_ref[...]
        h = jnp.maximum(h, 0.0)
        o_ref[...] = jnp.dot(h, w2_ref[...],
                             preferred_element_type=jnp.float32) + b2_ref[...]

    return pl.pallas_call(
        body,
        out_shape=jax.ShapeDtypeStruct((BATCH, PROMPT), jnp.float32),
        grid=(BATCH // BLK,),
        in_specs=[
            pl.BlockSpec((BLK, EMBED), lambda i: (i, 0)),
            pl.BlockSpec((EMBED, PROMPT), lambda i: (0, 0)),
            pl.BlockSpec((1, PROMPT), lambda i: (0, 0)),
            pl.BlockSpec((PROMPT, PROMPT), lambda i: (0, 0)),
            pl.BlockSpec((1, PROMPT), lambda i: (0, 0)),
        ],
        out_specs=pl.BlockSpec((BLK, PROMPT), lambda i: (i, 0)),
    )(pooled, W1, b1.reshape(1, PROMPT), W2, b2.reshape(1, PROMPT))


def kernel(prompt_ids, table, W1, b1, W2, b2):
    ids2d = prompt_ids.reshape(BATCH * SEQ // IDX_SLICE, IDX_SLICE)
    table_lin = _format_table(table.T).reshape(VOCAB, EMBED)
    pooled = _sc_gather_pool(ids2d, table_lin)
    return _mlp(pooled, W1, b1, W2, b2)


# speedup vs baseline: 2.2269x; 1.0135x over previous
import functools

import jax
import jax.numpy as jnp
from jax import lax
from jax.experimental import pallas as pl
from jax.experimental.pallas import tpu as pltpu
from jax.experimental.pallas import tpu_sc as plsc

VOCAB = 1000000
EMBED = 32
PROMPT = 128
BATCH = 16384
SEQ = 200

NC, NS = 2, 16
NW = NC * NS
B_PER_W = BATCH // NW
CB = 8
IDX_SLICE = 40
G_PER_CHUNK = CB * SEQ // IDX_SLICE
N_CHUNKS = B_PER_W // CB
IDX_ROWS_PER_W = B_PER_W * SEQ // IDX_SLICE
UNROLL = 8
INV_SEQ = 1.0 / SEQ


def _sc_gather_pool(ids2d, table):
    mesh = plsc.VectorSubcoreMesh(core_axis_name="c", subcore_axis_name="s")

    @functools.partial(
        pl.kernel,
        out_type=jax.ShapeDtypeStruct((BATCH, EMBED), jnp.float32),
        mesh=mesh,
        scratch_types=[
            pltpu.VMEM((3, G_PER_CHUNK, IDX_SLICE), jnp.int32),
            pltpu.VMEM((2, CB * SEQ, EMBED), jnp.float32),
            pltpu.VMEM((B_PER_W, EMBED), jnp.float32),
            pltpu.SemaphoreType.DMA,
            pltpu.SemaphoreType.DMA,
            pltpu.SemaphoreType.DMA,
            pltpu.SemaphoreType.DMA,
            pltpu.SemaphoreType.DMA,
        ],
        compiler_params=pltpu.CompilerParams(use_tc_tiling_on_sc=False),
    )
    def k(ids_hbm, table_hbm, out_hbm, idx_v, rows_v, pooled_v,
          sem0, sem1, isem0, isem1, isem2):
        wid = lax.axis_index("s") * NC + lax.axis_index("c")
        row0 = wid * IDX_ROWS_PER_W
        bbase = wid * B_PER_W
        sems = (sem0, sem1)
        isems = (isem0, isem1, isem2)
        last_row = (NW * IDX_ROWS_PER_W) - G_PER_CHUNK

        def idx_src(g):
            return ids_hbm.at[pl.ds(
                jnp.minimum(row0 + g * G_PER_CHUNK, last_row), G_PER_CHUNK)]

        def idx_fire(g, ibuf):
            pltpu.async_copy(idx_src(g), idx_v.at[ibuf], isems[ibuf])

        def idx_wait(g, ibuf):
            pltpu.make_async_copy(idx_src(g), idx_v.at[ibuf],
                                  isems[ibuf]).wait()

        def fire(ibuf, buf):
            for j in range(G_PER_CHUNK):
                pltpu.async_copy(
                    table_hbm.at[idx_v.at[ibuf, j]],
                    rows_v.at[buf, pl.ds(j * IDX_SLICE, IDX_SLICE)],
                    sems[buf])

        def drain(ibuf, buf):
            for j in range(G_PER_CHUNK):
                pltpu.make_async_copy(
                    table_hbm.at[idx_v.at[ibuf, j]],
                    rows_v.at[buf, pl.ds(j * IDX_SLICE, IDX_SLICE)],
                    sems[buf]).wait()

        def reduce_chunk(g, buf):
            for b in range(CB):
                zero = jnp.zeros((16,), jnp.float32)
                npairs = 4

                def body(i, accs, _b=b, _buf=buf):
                    accs = list(accs)
                    for u in range(UNROLL):
                        r = _b * SEQ + i * UNROLL + u
                        a = 2 * (u % npairs)
                        accs[a] = accs[a] + rows_v[_buf, r, pl.ds(0, 16)]
                        accs[a + 1] = (
                            accs[a + 1] + rows_v[_buf, r, pl.ds(16, 16)])
                    return tuple(accs)

                accs = lax.fori_loop(0, SEQ // UNROLL, body,
                                     (zero,) * (2 * npairs))
                a0 = (accs[0] + accs[2]) + (accs[4] + accs[6])
                a1 = (accs[1] + accs[3]) + (accs[5] + accs[7])
                pooled_v[g * CB + b, pl.ds(0, 16)] = a0 * INV_SEQ
                pooled_v[g * CB + b, pl.ds(16, 16)] = a1 * INV_SEQ

        assert (N_CHUNKS - 4) % 6 == 0
        idx_fire(0, 0)
        idx_fire(1, 1)
        idx_wait(0, 0)
        fire(0, 0)
        idx_fire(2, 2)

        @pl.loop(1, N_CHUNKS - 3, step=6)
        def _(go):
            for p in range(6):
                g = go + p
                rb = (1 + p) % 2
                ib = (1 + p) % 3
                ibm = (p) % 3
                idx_wait(g, ib)
                fire(ib, rb)
                drain(ibm, 1 - rb)
                idx_fire(g + 2, ibm)
                reduce_chunk(g - 1, 1 - rb)

        for g in range(N_CHUNKS - 3, N_CHUNKS):
            rb, ib, ibm = g % 2, g % 3, (g - 1) % 3
            idx_wait(g, ib)
            fire(ib, rb)
            drain(ibm, 1 - rb)
            if g + 2 < N_CHUNKS:
                idx_fire(g + 2, ibm)
            reduce_chunk(g - 1, 1 - rb)
        drain((N_CHUNKS - 1) % 3, (N_CHUNKS - 1) % 2)
        reduce_chunk(N_CHUNKS - 1, (N_CHUNKS - 1) % 2)

        pltpu.sync_copy(pooled_v, out_hbm.at[pl.ds(bbase, B_PER_W)])

    return k(ids2d, table)


def _format_table(tableT):
    VB = 16384
    SUB = 16
    VS = VB // SUB

    def body(x_ref, o_ref, t_scr):
        for s in range(SUB):
            t_scr[pl.ds(s * VS, VS), :] = jnp.transpose(
                x_ref[:, pl.ds(s * VS, VS)], (1, 0))
        for s in range(SUB):
            o_ref[pl.ds(s * VS // 4, VS // 4), :] = jnp.concatenate(
                [t_scr[pl.Slice(s * VS + q, VS // 4, 4), :]
                 for q in range(4)], axis=1)

    return pl.pallas_call(
        body,
        out_shape=jax.ShapeDtypeStruct((VOCAB // 4, 128), jnp.float32),
        grid=(pl.cdiv(VOCAB, VB),),
        in_specs=[pl.BlockSpec((EMBED, VB), lambda i: (0, i))],
        out_specs=pl.BlockSpec((VB // 4, 128), lambda i: (i, 0)),
        scratch_shapes=[pltpu.VMEM((VB, EMBED), jnp.float32)],
    )(tableT)


N_CBLK = VOCAB // 128
CBLK_PER_W = N_CBLK // NW
TAIL_V = N_CBLK * 128


def _sc_format(tableT, tail_lin):
    mesh = plsc.VectorSubcoreMesh(core_axis_name="c", subcore_axis_name="s")

    @functools.partial(
        pl.kernel,
        out_type=jax.ShapeDtypeStruct((VOCAB // 4, 128), jnp.float32),
        mesh=mesh,
        scratch_types=[
            pltpu.VMEM((2, 4, 8, 129), jnp.float32),
            pltpu.VMEM((2, 32, 128), jnp.float32),
            pltpu.VMEM((8, 8, EMBED), jnp.float32),
            pltpu.SemaphoreType.DMA,
            pltpu.SemaphoreType.DMA,
            pltpu.SemaphoreType.DMA,
            pltpu.SemaphoreType.DMA,
        ],
        compiler_params=pltpu.CompilerParams(use_tc_tiling_on_sc=True,
                                             needs_layout_passes=False),
    )
    def k(tab_hbm, tail_hbm, fmt_hbm, in_v, out_v, tail_v,
          is0, is1, os0, os1):
        wid = lax.axis_index("s") * NC + lax.axis_index("c")
        cb0 = wid * CBLK_PER_W
        isems = (is0, is1)
        osems = (os0, os1)
        iota = lax.iota(jnp.int32, 16)

        def in_pairs(cb, buf):
            return [(tab_hbm.at[pl.ds(8 * a, 8), pl.ds(cb * 128, 128)],
                     in_v.at[buf, a, :, pl.ds(0, 128)]) for a in range(4)]

        def in_fire(cb, buf):
            for s, d in in_pairs(cb, buf):
                pltpu.async_copy(s, d, isems[buf])

        def in_wait(cb, buf):
            for s, d in in_pairs(cb, buf):
                pltpu.make_async_copy(s, d, isems[buf]).wait()

        def out_fire(cb, buf):
            pltpu.async_copy(out_v.at[buf],
                             fmt_hbm.at[pl.ds(cb * 32, 32)], osems[buf])

        def out_wait(cb, buf):
            pltpu.make_async_copy(out_v.at[buf],
                                  fmt_hbm.at[pl.ds(cb * 32, 32)],
                                  osems[buf]).wait()

        def transpose(buf):
            src = in_v.at[buf]

            def body(rb, carry, _buf=buf):
                for u in range(8):
                    r = rb * 8 + u
                    for h in range(2):
                        vals = plsc.load_gather(
                            src,
                            [2 * h + iota // 8, iota % 8,
                             jnp.full((16,), r, jnp.int32)])
                        out_v[_buf, rb * 2 + u // 4,
                              pl.ds(32 * (u % 4) + 16 * h, 16)] = vals
                return carry

            lax.fori_loop(0, 16, body, 0)

        in_fire(cb0, 0)
        in_fire(cb0 + 1, 1)
        for p in range(2):
            in_wait(cb0 + p, p)
            transpose(p)
            out_fire(cb0 + p, p)
            in_fire(cb0 + p + 2, p)

        @pl.loop(2, CBLK_PER_W - 2, step=2)
        def _(go):
            for p in range(2):
                cb = cb0 + go + p
                in_wait(cb, p)
                out_wait(cb - 2, p)
                transpose(p)
                out_fire(cb, p)
                in_fire(cb + 2, p)

        for p in range(2):
            cb = cb0 + CBLK_PER_W - 2 + p
            in_wait(cb, p)
            out_wait(cb - 2, p)
            transpose(p)
            out_fire(cb, p)
        for p in range(2):
            out_wait(cb0 + CBLK_PER_W - 2 + p, p)

        @pl.when(wid < 4)
        def _():
            cb = NW * CBLK_PER_W + wid
            for s, d in in_pairs(cb, 0):
                pltpu.sync_copy(s, d)
            transpose(0)
            pltpu.sync_copy(out_v.at[0], fmt_hbm.at[pl.ds(cb * 32, 32)])

        @pl.when(wid == 4)
        def _():
            for a in range(8):
                pltpu.sync_copy(tail_hbm.at[pl.ds(8 * a, 8)], tail_v.at[a])
            for r in range(64):
                for h in range(2):
                    out_v[0, r // 4, pl.ds(32 * (r % 4) + 16 * h, 16)] = (
                        tail_v[r // 8, r % 8, pl.ds(16 * h, 16)])
            pltpu.sync_copy(out_v.at[0, pl.ds(0, 16)],
                            fmt_hbm.at[pl.ds(TAIL_V // 4, 16)])

    return k(tableT, tail_lin)


def _mlp(pooled, W1, b1, W2, b2):
    BLK = 2048

    def body(p_ref, w1_ref, b1_ref, w2_ref, b2_ref, o_ref):
        h = jnp.dot(p_ref[...], w1_ref[...],
                    preferred_element_type=jnp.float32) + b1_ref[...]
        h = jnp.maximum(h, 0.0)
        o_ref[...] = jnp.dot(h, w2_ref[...],
                             preferred_element_type=jnp.float32) + b2_ref[...]

    return pl.pallas_call(
        body,
        out_shape=jax.ShapeDtypeStruct((BATCH, PROMPT), jnp.float32),
        grid=(BATCH // BLK,),
        in_specs=[
            pl.BlockSpec((BLK, EMBED), lambda i: (i, 0)),
            pl.BlockSpec((EMBED, PROMPT), lambda i: (0, 0)),
            pl.BlockSpec((1, PROMPT), lambda i: (0, 0)),
            pl.BlockSpec((PROMPT, PROMPT), lambda i: (0, 0)),
            pl.BlockSpec((1, PROMPT), lambda i: (0, 0)),
        ],
        out_specs=pl.BlockSpec((BLK, PROMPT), lambda i: (i, 0)),
    )(pooled, W1, b1.reshape(1, PROMPT), W2, b2.reshape(1, PROMPT))


def kernel(prompt_ids, table, W1, b1, W2, b2):
    ids2d = prompt_ids.reshape(BATCH * SEQ // IDX_SLICE, IDX_SLICE)
    table_lin = _format_table(table.T).reshape(VOCAB, EMBED)
    pooled = _sc_gather_pool(ids2d, table_lin)
    return _mlp(pooled, W1, b1, W2, b2)


# speedup vs baseline: 2.2405x; 1.0061x over previous
import functools

import jax
import jax.numpy as jnp
from jax import lax
from jax.experimental import pallas as pl
from jax.experimental.pallas import tpu as pltpu
from jax.experimental.pallas import tpu_sc as plsc

VOCAB = 1000000
EMBED = 32
PROMPT = 128
BATCH = 16384
SEQ = 200

NC, NS = 2, 16
NW = NC * NS
B_PER_W = BATCH // NW
CB = 8
IDX_SLICE = 40
G_PER_CHUNK = CB * SEQ // IDX_SLICE
N_CHUNKS = B_PER_W // CB
IDX_ROWS_PER_W = B_PER_W * SEQ // IDX_SLICE
UNROLL = 8
INV_SEQ = 1.0 / SEQ


def _sc_gather_pool(ids2d, table):
    mesh = plsc.VectorSubcoreMesh(core_axis_name="c", subcore_axis_name="s")

    @functools.partial(
        pl.kernel,
        out_type=jax.ShapeDtypeStruct((BATCH, EMBED), jnp.float32),
        mesh=mesh,
        scratch_types=[
            pltpu.VMEM((3, G_PER_CHUNK, IDX_SLICE), jnp.int32),
            pltpu.VMEM((2, CB * SEQ, EMBED), jnp.float32),
            pltpu.VMEM((B_PER_W, EMBED), jnp.float32),
            pltpu.SemaphoreType.DMA,
            pltpu.SemaphoreType.DMA,
            pltpu.SemaphoreType.DMA,
            pltpu.SemaphoreType.DMA,
            pltpu.SemaphoreType.DMA,
        ],
        compiler_params=pltpu.CompilerParams(use_tc_tiling_on_sc=False),
    )
    def k(ids_hbm, table_hbm, out_hbm, idx_v, rows_v, pooled_v,
          sem0, sem1, isem0, isem1, isem2):
        wid = lax.axis_index("s") * NC + lax.axis_index("c")
        row0 = wid * IDX_ROWS_PER_W
        bbase = wid * B_PER_W
        sems = (sem0, sem1)
        isems = (isem0, isem1, isem2)
        last_row = (NW * IDX_ROWS_PER_W) - G_PER_CHUNK

        def idx_src(g):
            return ids_hbm.at[pl.ds(
                jnp.minimum(row0 + g * G_PER_CHUNK, last_row), G_PER_CHUNK)]

        def idx_fire(g, ibuf):
            pltpu.async_copy(idx_src(g), idx_v.at[ibuf], isems[ibuf])

        def idx_wait(g, ibuf):
            pltpu.make_async_copy(idx_src(g), idx_v.at[ibuf],
                                  isems[ibuf]).wait()

        def fire(ibuf, buf):
            for j in range(G_PER_CHUNK):
                pltpu.async_copy(
                    table_hbm.at[idx_v.at[ibuf, j]],
                    rows_v.at[buf, pl.ds(j * IDX_SLICE, IDX_SLICE)],
                    sems[buf])

        def drain(ibuf, buf):
            for j in range(G_PER_CHUNK):
                pltpu.make_async_copy(
                    table_hbm.at[idx_v.at[ibuf, j]],
                    rows_v.at[buf, pl.ds(j * IDX_SLICE, IDX_SLICE)],
                    sems[buf]).wait()

        def reduce_chunk(g, buf):
            for b in range(CB):
                zero = jnp.zeros((16,), jnp.float32)
                npairs = 4

                def body(i, accs, _b=b, _buf=buf):
                    accs = list(accs)
                    for u in range(UNROLL):
                        r = _b * SEQ + i * UNROLL + u
                        a = 2 * (u % npairs)
                        accs[a] = accs[a] + rows_v[_buf, r, pl.ds(0, 16)]
                        accs[a + 1] = (
                            accs[a + 1] + rows_v[_buf, r, pl.ds(16, 16)])
                    return tuple(accs)

                accs = lax.fori_loop(0, SEQ // UNROLL, body,
                                     (zero,) * (2 * npairs))
                a0 = (accs[0] + accs[2]) + (accs[4] + accs[6])
                a1 = (accs[1] + accs[3]) + (accs[5] + accs[7])
                pooled_v[g * CB + b, pl.ds(0, 16)] = a0 * INV_SEQ
                pooled_v[g * CB + b, pl.ds(16, 16)] = a1 * INV_SEQ

        assert (N_CHUNKS - 4) % 6 == 0
        idx_fire(0, 0)
        idx_fire(1, 1)
        idx_wait(0, 0)
        fire(0, 0)
        idx_fire(2, 2)

        @pl.loop(1, N_CHUNKS - 3, step=6)
        def _(go):
            for p in range(6):
                g = go + p
                rb = (1 + p) % 2
                ib = (1 + p) % 3
                ibm = (p) % 3
                idx_wait(g, ib)
                fire(ib, rb)
                drain(ibm, 1 - rb)
                idx_fire(g + 2, ibm)
                reduce_chunk(g - 1, 1 - rb)

        for g in range(N_CHUNKS - 3, N_CHUNKS):
            rb, ib, ibm = g % 2, g % 3, (g - 1) % 3
            idx_wait(g, ib)
            fire(ib, rb)
            drain(ibm, 1 - rb)
            if g + 2 < N_CHUNKS:
                idx_fire(g + 2, ibm)
            reduce_chunk(g - 1, 1 - rb)
        drain((N_CHUNKS - 1) % 3, (N_CHUNKS - 1) % 2)
        reduce_chunk(N_CHUNKS - 1, (N_CHUNKS - 1) % 2)

        pltpu.sync_copy(pooled_v, out_hbm.at[pl.ds(bbase, B_PER_W)])

    return k(ids2d, table)


def _format_table(tableT):
    VB = 32768
    SUB = 32
    VS = VB // SUB

    def body(x_ref, o_ref, t_scr):
        for s in range(SUB):
            t_scr[pl.ds(s * VS, VS), :] = jnp.transpose(
                x_ref[:, pl.ds(s * VS, VS)], (1, 0))
        for s in range(SUB):
            o_ref[pl.ds(s * VS // 4, VS // 4), :] = jnp.concatenate(
                [t_scr[pl.Slice(s * VS + q, VS // 4, 4), :]
                 for q in range(4)], axis=1)

    return pl.pallas_call(
        body,
        out_shape=jax.ShapeDtypeStruct((VOCAB // 4, 128), jnp.float32),
        grid=(pl.cdiv(VOCAB, VB),),
        in_specs=[pl.BlockSpec((EMBED, VB), lambda i: (0, i))],
        out_specs=pl.BlockSpec((VB // 4, 128), lambda i: (i, 0)),
        scratch_shapes=[pltpu.VMEM((VB, EMBED), jnp.float32)],
    )(tableT)


N_CBLK = VOCAB // 128
CBLK_PER_W = N_CBLK // NW
TAIL_V = N_CBLK * 128


def _sc_format(tableT, tail_lin):
    mesh = plsc.VectorSubcoreMesh(core_axis_name="c", subcore_axis_name="s")

    @functools.partial(
        pl.kernel,
        out_type=jax.ShapeDtypeStruct((VOCAB // 4, 128), jnp.float32),
        mesh=mesh,
        scratch_types=[
            pltpu.VMEM((2, 4, 8, 129), jnp.float32),
            pltpu.VMEM((2, 32, 128), jnp.float32),
            pltpu.VMEM((8, 8, EMBED), jnp.float32),
            pltpu.SemaphoreType.DMA,
            pltpu.SemaphoreType.DMA,
            pltpu.SemaphoreType.DMA,
            pltpu.SemaphoreType.DMA,
        ],
        compiler_params=pltpu.CompilerParams(use_tc_tiling_on_sc=True,
                                             needs_layout_passes=False),
    )
    def k(tab_hbm, tail_hbm, fmt_hbm, in_v, out_v, tail_v,
          is0, is1, os0, os1):
        wid = lax.axis_index("s") * NC + lax.axis_index("c")
        cb0 = wid * CBLK_PER_W
        isems = (is0, is1)
        osems = (os0, os1)
        iota = lax.iota(jnp.int32, 16)

        def in_pairs(cb, buf):
            return [(tab_hbm.at[pl.ds(8 * a, 8), pl.ds(cb * 128, 128)],
                     in_v.at[buf, a, :, pl.ds(0, 128)]) for a in range(4)]

        def in_fire(cb, buf):
            for s, d in in_pairs(cb, buf):
                pltpu.async_copy(s, d, isems[buf])

        def in_wait(cb, buf):
            for s, d in in_pairs(cb, buf):
                pltpu.make_async_copy(s, d, isems[buf]).wait()

        def out_fire(cb, buf):
            pltpu.async_copy(out_v.at[buf],
                             fmt_hbm.at[pl.ds(cb * 32, 32)], osems[buf])

        def out_wait(cb, buf):
            pltpu.make_async_copy(out_v.at[buf],
                                  fmt_hbm.at[pl.ds(cb * 32, 32)],
                                  osems[buf]).wait()

        def transpose(buf):
            src = in_v.at[buf]

            def body(rb, carry, _buf=buf):
                for u in range(8):
                    r = rb * 8 + u
                    for h in range(2):
                        vals = plsc.load_gather(
                            src,
                            [2 * h + iota // 8, iota % 8,
                             jnp.full((16,), r, jnp.int32)])
                        out_v[_buf, rb * 2 + u // 4,
                              pl.ds(32 * (u % 4) + 16 * h, 16)] = vals
                return carry

            lax.fori_loop(0, 16, body, 0)

        in_fire(cb0, 0)
        in_fire(cb0 + 1, 1)
        for p in range(2):
            in_wait(cb0 + p, p)
            transpose(p)
            out_fire(cb0 + p, p)
            in_fire(cb0 + p + 2, p)

        @pl.loop(2, CBLK_PER_W - 2, step=2)
        def _(go):
            for p in range(2):
                cb = cb0 + go + p
                in_wait(cb, p)
                out_wait(cb - 2, p)
                transpose(p)
                out_fire(cb, p)
                in_fire(cb + 2, p)

        for p in range(2):
            cb = cb0 + CBLK_PER_W - 2 + p
            in_wait(cb, p)
            out_wait(cb - 2, p)
            transpose(p)
            out_fire(cb, p)
        for p in range(2):
            out_wait(cb0 + CBLK_PER_W - 2 + p, p)

        @pl.when(wid < 4)
        def _():
            cb = NW * CBLK_PER_W + wid
            for s, d in in_pairs(cb, 0):
                pltpu.sync_copy(s, d)
            transpose(0)
            pltpu.sync_copy(out_v.at[0], fmt_hbm.at[pl.ds(cb * 32, 32)])

        @pl.when(wid == 4)
        def _():
            for a in range(8):
                pltpu.sync_copy(tail_hbm.at[pl.ds(8 * a, 8)], tail_v.at[a])
            for r in range(64):
                for h in range(2):
                    out_v[0, r // 4, pl.ds(32 * (r % 4) + 16 * h, 16)] = (
                        tail_v[r // 8, r % 8, pl.ds(16 * h, 16)])
            pltpu.sync_copy(out_v.at[0, pl.ds(0, 16)],
                            fmt_hbm.at[pl.ds(TAIL_V // 4, 16)])

    return k(tableT, tail_lin)


def _mlp(pooled, W1, b1, W2, b2):
    BLK = 2048

    def body(p_ref, w1_ref, b1_ref, w2_ref, b2_ref, o_ref):
        h = jnp.dot(p_ref[...], w1_ref[...],
                    preferred_element_type=jnp.float32) + b1_ref[...]
        h = jnp.maximum(h, 0.0)
        o_ref[...] = jnp.dot(h, w2_ref[...],
                             preferred_element_type=jnp.float32) + b2_ref[...]

    return pl.pallas_call(
        body,
        out_shape=jax.ShapeDtypeStruct((BATCH, PROMPT), jnp.float32),
        grid=(BATCH // BLK,),
        in_specs=[
            pl.BlockSpec((BLK, EMBED), lambda i: (i, 0)),
            pl.BlockSpec((EMBED, PROMPT), lambda i: (0, 0)),
            pl.BlockSpec((1, PROMPT), lambda i: (0, 0)),
            pl.BlockSpec((PROMPT, PROMPT), lambda i: (0, 0)),
            pl.BlockSpec((1, PROMPT), lambda i: (0, 0)),
        ],
        out_specs=pl.BlockSpec((BLK, PROMPT), lambda i: (i, 0)),
    )(pooled, W1, b1.reshape(1, PROMPT), W2, b2.reshape(1, PROMPT))


def kernel(prompt_ids, table, W1, b1, W2, b2):
    ids2d = prompt_ids.reshape(BATCH * SEQ // IDX_SLICE, IDX_SLICE)
    table_lin = _format_table(table.T).reshape(VOCAB, EMBED)
    pooled = _sc_gather_pool(ids2d, table_lin)
    return _mlp(pooled, W1, b1, W2, b2)
